# baseline (speedup 1.0000x reference)
import functools

import jax
import jax.numpy as jnp
from jax import lax
from jax.experimental import pallas as pl
from jax.experimental.pallas import tpu as pltpu
from jax.experimental.pallas import tpu_sc as plsc

_SEGS = 64
_N = 50000
_D = 1024

_NW = 32
_RB = 16
_NS = 16384
_CH = _NS // _NW
_KW = _CH // _RB

_BT = 2048
_TC0 = _NS // _BT
_NTB = -(-(_N - _NS) // _BT)
_NTPAD = _NTB * _BT


_sc_mesh = plsc.VectorSubcoreMesh(core_axis_name="c", subcore_axis_name="s")


@functools.partial(
    pl.kernel,
    out_type=(
        jax.ShapeDtypeStruct((_NW * _SEGS, _D), jnp.float32),
        jax.ShapeDtypeStruct((_NW * _SEGS,), jnp.float32),
    ),
    mesh=_sc_mesh,
    compiler_params=pltpu.CompilerParams(needs_layout_passes=False),
    scratch_types=[
        pltpu.VMEM((_RB, _D), jnp.float32),
        pltpu.VMEM((_RB, _D), jnp.float32),
        pltpu.VMEM((_CH,), jnp.int32),
        pltpu.VMEM((_SEGS,), jnp.float32),
        pltpu.VMEM((_SEGS, _D), jnp.float32),
        pltpu.SemaphoreType.DMA,
    ],
)
def _sc_segsum(x_hbm, batch_hbm, part_hbm, cnt_hbm,
               xbuf0_v, xbuf1_v, bv_v, cnt_v, acc_v, sem):
    cid = lax.axis_index("c")
    sid = lax.axis_index("s")
    wid = cid * 16 + sid
    base = wid * _CH

    pltpu.sync_copy(batch_hbm.at[pl.ds(base, _CH)], bv_v)

    zeros16 = jnp.zeros((16,), jnp.float32)
    for c in range(_SEGS // 16):
        cnt_v[pl.ds(c * 16, 16)] = zeros16

    def zbody(s, carry):
        for c in range(_D // 16):
            acc_v[s, pl.ds(c * 16, 16)] = zeros16
        return carry

    lax.fori_loop(0, _SEGS, zbody, 0)

    ones16 = jnp.ones((16,), jnp.float32)

    def win_copy(k, buf):
        return pltpu.make_async_copy(
            x_hbm.at[pl.ds(base + k * _RB, _RB), :], buf, sem)

    def process(k, buf):
        vidx = bv_v[pl.ds(k * _RB, _RB)]
        plsc.addupdate_scatter(cnt_v, [vidx], ones16)
        seg0 = vidx[0]
        uniform = seg0 == vidx[_RB - 1]

        @pl.when(uniform)
        def _():
            for c in range(_D // 16):
                vals = [buf[r, pl.ds(c * 16, 16)] for r in range(_RB)]
                while len(vals) > 1:
                    vals = [vals[i] + vals[i + 1]
                            for i in range(0, len(vals), 2)]
                plsc.addupdate(acc_v.at[seg0, pl.ds(c * 16, 16)], vals[0])

        @pl.when(jnp.logical_not(uniform))
        def _():
            for r in range(_RB):
                seg = vidx[r]
                for c in range(_D // 16):
                    plsc.addupdate(acc_v.at[seg, pl.ds(c * 16, 16)],
                                   buf[r, pl.ds(c * 16, 16)])

    win_copy(0, xbuf0_v).start()

    def body(j, carry):
        k = j * 2
        win_copy(k, xbuf0_v).wait()
        win_copy(k + 1, xbuf1_v).start()
        process(k, xbuf0_v)
        win_copy(k + 1, xbuf1_v).wait()

        @pl.when(k + 2 < _KW)
        def _():
            win_copy(k + 2, xbuf0_v).start()

        process(k + 1, xbuf1_v)
        return carry

    lax.fori_loop(0, _KW // 2, body, 0)

    pltpu.sync_copy(acc_v, part_hbm.at[pl.ds(wid * _SEGS, _SEGS), :])
    pltpu.sync_copy(cnt_v, cnt_hbm.at[pl.ds(wid * _SEGS, _SEGS)])


def _tc_body(x_ref, batch_ref, w_ref, sums_ref, cnt_ref):
    i = pl.program_id(0)

    @pl.when(i == 0)
    def _():
        sums_ref[...] = jnp.zeros_like(sums_ref)
        cnt_ref[...] = jnp.zeros_like(cnt_ref)

    h2 = lax.dot_general(w_ref[...], x_ref[...], (((1,), (1,)), ((), ())),
                         preferred_element_type=jnp.float32)
    row = lax.broadcasted_iota(jnp.int32, (1, _BT), 1) + (_NS + i * _BT)
    h2 = jnp.where(row < _N, h2, 0.0)
    bidx = batch_ref[0]
    seg = lax.broadcasted_iota(jnp.int32, (_SEGS, _BT), 0)
    onehot = (bidx == seg).astype(jnp.float32)
    psum = lax.dot_general(h2, onehot, (((1,), (1,)), ((), ())),
                           preferred_element_type=jnp.float32)
    ones = jnp.ones((1, _BT), jnp.float32)
    pcnt = lax.dot_general(ones, onehot, (((1,), (1,)), ((), ())),
                           preferred_element_type=jnp.float32)
    sums_ref[...] += psum
    cnt_ref[...] += pcnt


def _tc_partial(x, batch_tc3, W):
    return pl.pallas_call(
        _tc_body,
        grid=(_NTB,),
        in_specs=[
            pl.BlockSpec((_BT, _D), lambda i: (_TC0 + i, 0)),
            pl.BlockSpec((1, 1, _BT), lambda i: (i, 0, 0)),
            pl.BlockSpec((2, _D), lambda i: (0, 0)),
        ],
        out_specs=[
            pl.BlockSpec((2, _SEGS), lambda i: (0, 0)),
            pl.BlockSpec((1, _SEGS), lambda i: (0, 0)),
        ],
        out_shape=[
            jax.ShapeDtypeStruct((2, _SEGS), jnp.float32),
            jax.ShapeDtypeStruct((1, _SEGS), jnp.float32),
        ],
        compiler_params=pltpu.CompilerParams(
            dimension_semantics=("arbitrary",)),
    )(x, batch_tc3, W)


def _combine_body(part_ref, cntsc_ref, tcs_ref, tcc_ref, w_ref, b_ref,
                  out_ref):
    s = part_ref[pl.ds(0, _SEGS), :]
    for t in range(1, _NW):
        s = s + part_ref[pl.ds(t * _SEGS, _SEGS), :]
    hsc = lax.dot_general(w_ref[...], s, (((1,), (1,)), ((), ())),
                          preferred_element_type=jnp.float32)
    c = cntsc_ref[pl.ds(0, _SEGS)]
    for t in range(1, _NW):
        c = c + cntsc_ref[pl.ds(t * _SEGS, _SEGS)]
    cnt = c[None, :] + tcc_ref[...]
    out_ref[...] = (hsc + tcs_ref[...] + cnt * b_ref[...]) / \
        jnp.maximum(cnt, 1.0)


def _combine(partials, cnt_sc, tc_sums, tc_cnt, W, b2):
    return pl.pallas_call(
        _combine_body,
        in_specs=[
            pl.BlockSpec((_NW * _SEGS, _D), lambda: (0, 0)),
            pl.BlockSpec((_NW * _SEGS,), lambda: (0,)),
            pl.BlockSpec((2, _SEGS), lambda: (0, 0)),
            pl.BlockSpec((1, _SEGS), lambda: (0, 0)),
            pl.BlockSpec((2, _D), lambda: (0, 0)),
            pl.BlockSpec((2, 1), lambda: (0, 0)),
        ],
        out_specs=pl.BlockSpec((2, _SEGS), lambda: (0, 0)),
        out_shape=jax.ShapeDtypeStruct((2, _SEGS), jnp.float32),
    )(partials, cnt_sc, tc_sums, tc_cnt, W, b2)


def kernel(x, edge_index, batch, W, b):
    partials, cnt_sc = _sc_segsum(x, batch)
    batch_tc = jnp.concatenate(
        [batch[_NS:], jnp.full((_NTPAD - (_N - _NS),), _SEGS, jnp.int32)])
    tc_sums, tc_cnt = _tc_partial(x, batch_tc.reshape(_NTB, 1, _BT), W)
    out2 = _combine(partials, cnt_sc, tc_sums, tc_cnt, W, b.reshape(2, 1))
    return out2.T


# speedup vs baseline: 1.4784x; 1.4784x over previous
import functools

import jax
import jax.numpy as jnp
from jax import lax
from jax.experimental import pallas as pl
from jax.experimental.pallas import tpu as pltpu
from jax.experimental.pallas import tpu_sc as plsc

_SEGS = 64
_N = 50000
_D = 1024

_NW = 32
_RB = 16
_NS = 10240
_CH = _NS // _NW
_KW = _CH // _RB

_BT = 2048
_TC0 = _NS // _BT
_NTB = -(-(_N - _NS) // _BT)
_NTPAD = _NTB * _BT


_sc_mesh = plsc.VectorSubcoreMesh(core_axis_name="c", subcore_axis_name="s")


@functools.partial(
    pl.kernel,
    out_type=(
        jax.ShapeDtypeStruct((_NW * _SEGS, _D), jnp.float32),
        jax.ShapeDtypeStruct((_NW * _SEGS,), jnp.float32),
    ),
    mesh=_sc_mesh,
    compiler_params=pltpu.CompilerParams(needs_layout_passes=False),
    scratch_types=[
        pltpu.VMEM((2, _RB, _D), jnp.float32),
        pltpu.VMEM((_CH,), jnp.int32),
        pltpu.VMEM((_SEGS,), jnp.float32),
        pltpu.VMEM((_SEGS, _D), jnp.float32),
        pltpu.SemaphoreType.DMA,
    ],
)
def _sc_segsum(x_hbm, batch_hbm, part_hbm, cnt_hbm,
               xbuf_v, bv_v, cnt_v, acc_v, sem):
    cid = lax.axis_index("c")
    sid = lax.axis_index("s")
    wid = cid * 16 + sid
    base = wid * _CH

    pltpu.sync_copy(batch_hbm.at[pl.ds(base, _CH)], bv_v)

    zeros16 = jnp.zeros((16,), jnp.float32)
    for c in range(_SEGS // 16):
        cnt_v[pl.ds(c * 16, 16)] = zeros16

    def zbody(s, carry):
        for c in range(_D // 16):
            acc_v[s, pl.ds(c * 16, 16)] = zeros16
        return carry

    lax.fori_loop(0, _SEGS, zbody, 0)

    ones16 = jnp.ones((16,), jnp.float32)

    def win_copy(k, slot):
        return pltpu.make_async_copy(
            x_hbm.at[pl.ds(base + k * _RB, _RB), :], xbuf_v.at[slot], sem)

    win_copy(0, 0).start()

    def body(k, carry):
        slot = lax.rem(k, 2)
        win_copy(k, slot).wait()

        @pl.when(k + 1 < _KW)
        def _():
            win_copy(k + 1, 1 - slot).start()

        vidx = bv_v[pl.ds(k * _RB, _RB)]
        plsc.addupdate_scatter(cnt_v, [vidx], ones16)
        seg0 = vidx[0]
        uniform = seg0 == vidx[_RB - 1]

        @pl.when(uniform)
        def _():
            for c in range(_D // 16):
                vals = [xbuf_v[slot, r, pl.ds(c * 16, 16)]
                        for r in range(_RB)]
                while len(vals) > 1:
                    vals = [vals[i] + vals[i + 1]
                            for i in range(0, len(vals), 2)]
                plsc.addupdate(acc_v.at[seg0, pl.ds(c * 16, 16)], vals[0])

        @pl.when(jnp.logical_not(uniform))
        def _():
            for r in range(_RB):
                seg = vidx[r]
                for c in range(_D // 16):
                    plsc.addupdate(acc_v.at[seg, pl.ds(c * 16, 16)],
                                   xbuf_v[slot, r, pl.ds(c * 16, 16)])
        return carry

    lax.fori_loop(0, _KW, body, 0)

    pltpu.sync_copy(acc_v, part_hbm.at[pl.ds(wid * _SEGS, _SEGS), :])
    pltpu.sync_copy(cnt_v, cnt_hbm.at[pl.ds(wid * _SEGS, _SEGS)])


def _tc_body(x_ref, batch_ref, w_ref, sums_ref, cnt_ref):
    i = pl.program_id(0)

    @pl.when(i == 0)
    def _():
        sums_ref[...] = jnp.zeros_like(sums_ref)
        cnt_ref[...] = jnp.zeros_like(cnt_ref)

    h2 = lax.dot_general(w_ref[...], x_ref[...], (((1,), (1,)), ((), ())),
                         preferred_element_type=jnp.float32)
    row = lax.broadcasted_iota(jnp.int32, (1, _BT), 1) + (_NS + i * _BT)
    h2 = jnp.where(row < _N, h2, 0.0)
    bidx = batch_ref[0]
    seg = lax.broadcasted_iota(jnp.int32, (_SEGS, _BT), 0)
    onehot = (bidx == seg).astype(jnp.float32)
    psum = lax.dot_general(h2, onehot, (((1,), (1,)), ((), ())),
                           preferred_element_type=jnp.float32)
    ones = jnp.ones((1, _BT), jnp.float32)
    pcnt = lax.dot_general(ones, onehot, (((1,), (1,)), ((), ())),
                           preferred_element_type=jnp.float32)
    sums_ref[...] += psum
    cnt_ref[...] += pcnt


def _tc_partial(x, batch_tc3, W):
    return pl.pallas_call(
        _tc_body,
        grid=(_NTB,),
        in_specs=[
            pl.BlockSpec((_BT, _D), lambda i: (_TC0 + i, 0)),
            pl.BlockSpec((1, 1, _BT), lambda i: (i, 0, 0)),
            pl.BlockSpec((2, _D), lambda i: (0, 0)),
        ],
        out_specs=[
            pl.BlockSpec((2, _SEGS), lambda i: (0, 0)),
            pl.BlockSpec((1, _SEGS), lambda i: (0, 0)),
        ],
        out_shape=[
            jax.ShapeDtypeStruct((2, _SEGS), jnp.float32),
            jax.ShapeDtypeStruct((1, _SEGS), jnp.float32),
        ],
        compiler_params=pltpu.CompilerParams(
            dimension_semantics=("arbitrary",)),
    )(x, batch_tc3, W)


def _combine_body(part_ref, cntsc_ref, tcs_ref, tcc_ref, w_ref, b_ref,
                  out_ref):
    s = part_ref[pl.ds(0, _SEGS), :]
    for t in range(1, _NW):
        s = s + part_ref[pl.ds(t * _SEGS, _SEGS), :]
    hsc = lax.dot_general(w_ref[...], s, (((1,), (1,)), ((), ())),
                          preferred_element_type=jnp.float32)
    c = cntsc_ref[pl.ds(0, _SEGS)]
    for t in range(1, _NW):
        c = c + cntsc_ref[pl.ds(t * _SEGS, _SEGS)]
    cnt = c[None, :] + tcc_ref[...]
    out_ref[...] = (hsc + tcs_ref[...] + cnt * b_ref[...]) / \
        jnp.maximum(cnt, 1.0)


def _combine(partials, cnt_sc, tc_sums, tc_cnt, W, b2):
    return pl.pallas_call(
        _combine_body,
        in_specs=[
            pl.BlockSpec((_NW * _SEGS, _D), lambda: (0, 0)),
            pl.BlockSpec((_NW * _SEGS,), lambda: (0,)),
            pl.BlockSpec((2, _SEGS), lambda: (0, 0)),
            pl.BlockSpec((1, _SEGS), lambda: (0, 0)),
            pl.BlockSpec((2, _D), lambda: (0, 0)),
            pl.BlockSpec((2, 1), lambda: (0, 0)),
        ],
        out_specs=pl.BlockSpec((2, _SEGS), lambda: (0, 0)),
        out_shape=jax.ShapeDtypeStruct((2, _SEGS), jnp.float32),
    )(partials, cnt_sc, tc_sums, tc_cnt, W, b2)


def kernel(x, edge_index, batch, W, b):
    partials, cnt_sc = _sc_segsum(x, batch)
    batch_tc = jnp.concatenate(
        [batch[_NS:], jnp.full((_NTPAD - (_N - _NS),), _SEGS, jnp.int32)])
    tc_sums, tc_cnt = _tc_partial(x, batch_tc.reshape(_NTB, 1, _BT), W)
    out2 = _combine(partials, cnt_sc, tc_sums, tc_cnt, W, b.reshape(2, 1))
    return out2.T


# speedup vs baseline: 1.4857x; 1.0049x over previous
import functools

import jax
import jax.numpy as jnp
from jax import lax
from jax.experimental import pallas as pl
from jax.experimental.pallas import tpu as pltpu
from jax.experimental.pallas import tpu_sc as plsc

_SEGS = 64
_N = 50000
_D = 1024

_NW = 32
_RB = 16
_NS = 10240
_CH = _NS // _NW
_KW = _CH // _RB

_BT = 2048
_TC0 = _NS // _BT
_NTB = -(-(_N - _NS) // _BT)
_NTPAD = _NTB * _BT


_sc_mesh = plsc.VectorSubcoreMesh(core_axis_name="c", subcore_axis_name="s")


@functools.partial(
    pl.kernel,
    out_type=(
        jax.ShapeDtypeStruct((_NW * _SEGS, _D), jnp.float32),
        jax.ShapeDtypeStruct((_NW * _SEGS,), jnp.float32),
    ),
    mesh=_sc_mesh,
    compiler_params=pltpu.CompilerParams(needs_layout_passes=False),
    scratch_types=[
        pltpu.VMEM((2, _RB, _D), jnp.float32),
        pltpu.VMEM((_CH,), jnp.int32),
        pltpu.VMEM((_SEGS,), jnp.float32),
        pltpu.VMEM((_SEGS, _D), jnp.float32),
        pltpu.SemaphoreType.DMA,
        pltpu.SemaphoreType.DMA,
    ],
)
def _sc_segsum(x_hbm, batch_hbm, part_hbm, cnt_hbm,
               xbuf_v, bv_v, cnt_v, acc_v, sem0, sem1):
    cid = lax.axis_index("c")
    sid = lax.axis_index("s")
    wid = cid * 16 + sid
    base = wid * _CH

    pltpu.sync_copy(batch_hbm.at[pl.ds(base, _CH)], bv_v)

    zeros16 = jnp.zeros((16,), jnp.float32)
    for c in range(_SEGS // 16):
        cnt_v[pl.ds(c * 16, 16)] = zeros16

    def zbody(s, carry):
        for c in range(_D // 16):
            acc_v[s, pl.ds(c * 16, 16)] = zeros16
        return carry

    lax.fori_loop(0, _SEGS, zbody, 0)

    ones16 = jnp.ones((16,), jnp.float32)

    def win_copy(k, slot, sem):
        return pltpu.make_async_copy(
            x_hbm.at[pl.ds(base + k * _RB, _RB), :], xbuf_v.at[slot], sem)

    win_copy(0, 0, sem0).start()

    def body(k, carry):
        slot = lax.rem(k, 2)
        even = slot == 0
        @pl.when(even)
        def _():
            win_copy(k, slot, sem0).wait()

        @pl.when(jnp.logical_not(even))
        def _():
            win_copy(k, slot, sem1).wait()

        @pl.when(jnp.logical_and(k + 1 < _KW, even))
        def _():
            win_copy(k + 1, 1 - slot, sem1).start()

        @pl.when(jnp.logical_and(k + 1 < _KW, jnp.logical_not(even)))
        def _():
            win_copy(k + 1, 1 - slot, sem0).start()

        vidx = bv_v[pl.ds(k * _RB, _RB)]
        plsc.addupdate_scatter(cnt_v, [vidx], ones16)
        seg0 = vidx[0]
        uniform = seg0 == vidx[_RB - 1]

        @pl.when(uniform)
        def _():
            for c in range(_D // 16):
                vals = [xbuf_v[slot, r, pl.ds(c * 16, 16)]
                        for r in range(_RB)]
                while len(vals) > 1:
                    vals = [vals[i] + vals[i + 1]
                            for i in range(0, len(vals), 2)]
                plsc.addupdate(acc_v.at[seg0, pl.ds(c * 16, 16)], vals[0])

        @pl.when(jnp.logical_not(uniform))
        def _():
            for r in range(_RB):
                seg = vidx[r]
                for c in range(_D // 16):
                    plsc.addupdate(acc_v.at[seg, pl.ds(c * 16, 16)],
                                   xbuf_v[slot, r, pl.ds(c * 16, 16)])
        return carry

    lax.fori_loop(0, _KW, body, 0)

    pltpu.sync_copy(acc_v, part_hbm.at[pl.ds(wid * _SEGS, _SEGS), :])
    pltpu.sync_copy(cnt_v, cnt_hbm.at[pl.ds(wid * _SEGS, _SEGS)])


def _tc_body(x_ref, batch_ref, w_ref, sums_ref, cnt_ref):
    i = pl.program_id(0)

    @pl.when(i == 0)
    def _():
        sums_ref[...] = jnp.zeros_like(sums_ref)
        cnt_ref[...] = jnp.zeros_like(cnt_ref)

    h2 = lax.dot_general(w_ref[...], x_ref[...], (((1,), (1,)), ((), ())),
                         preferred_element_type=jnp.float32)
    row = lax.broadcasted_iota(jnp.int32, (1, _BT), 1) + (_NS + i * _BT)
    h2 = jnp.where(row < _N, h2, 0.0)
    bidx = batch_ref[0]
    seg = lax.broadcasted_iota(jnp.int32, (_SEGS, _BT), 0)
    onehot = (bidx == seg).astype(jnp.float32)
    psum = lax.dot_general(h2, onehot, (((1,), (1,)), ((), ())),
                           preferred_element_type=jnp.float32)
    ones = jnp.ones((1, _BT), jnp.float32)
    pcnt = lax.dot_general(ones, onehot, (((1,), (1,)), ((), ())),
                           preferred_element_type=jnp.float32)
    sums_ref[...] += psum
    cnt_ref[...] += pcnt


def _tc_partial(x, batch_tc3, W):
    return pl.pallas_call(
        _tc_body,
        grid=(_NTB,),
        in_specs=[
            pl.BlockSpec((_BT, _D), lambda i: (_TC0 + i, 0)),
            pl.BlockSpec((1, 1, _BT), lambda i: (i, 0, 0)),
            pl.BlockSpec((2, _D), lambda i: (0, 0)),
        ],
        out_specs=[
            pl.BlockSpec((2, _SEGS), lambda i: (0, 0)),
            pl.BlockSpec((1, _SEGS), lambda i: (0, 0)),
        ],
        out_shape=[
            jax.ShapeDtypeStruct((2, _SEGS), jnp.float32),
            jax.ShapeDtypeStruct((1, _SEGS), jnp.float32),
        ],
        compiler_params=pltpu.CompilerParams(
            dimension_semantics=("arbitrary",)),
    )(x, batch_tc3, W)


def _combine_body(part_ref, cntsc_ref, tcs_ref, tcc_ref, w_ref, b_ref,
                  out_ref):
    s = part_ref[pl.ds(0, _SEGS), :]
    for t in range(1, _NW):
        s = s + part_ref[pl.ds(t * _SEGS, _SEGS), :]
    hsc = lax.dot_general(w_ref[...], s, (((1,), (1,)), ((), ())),
                          preferred_element_type=jnp.float32)
    c = cntsc_ref[pl.ds(0, _SEGS)]
    for t in range(1, _NW):
        c = c + cntsc_ref[pl.ds(t * _SEGS, _SEGS)]
    cnt = c[None, :] + tcc_ref[...]
    out_ref[...] = (hsc + tcs_ref[...] + cnt * b_ref[...]) / \
        jnp.maximum(cnt, 1.0)


def _combine(partials, cnt_sc, tc_sums, tc_cnt, W, b2):
    return pl.pallas_call(
        _combine_body,
        in_specs=[
            pl.BlockSpec((_NW * _SEGS, _D), lambda: (0, 0)),
            pl.BlockSpec((_NW * _SEGS,), lambda: (0,)),
            pl.BlockSpec((2, _SEGS), lambda: (0, 0)),
            pl.BlockSpec((1, _SEGS), lambda: (0, 0)),
            pl.BlockSpec((2, _D), lambda: (0, 0)),
            pl.BlockSpec((2, 1), lambda: (0, 0)),
        ],
        out_specs=pl.BlockSpec((2, _SEGS), lambda: (0, 0)),
        out_shape=jax.ShapeDtypeStruct((2, _SEGS), jnp.float32),
    )(partials, cnt_sc, tc_sums, tc_cnt, W, b2)


def kernel(x, edge_index, batch, W, b):
    partials, cnt_sc = _sc_segsum(x, batch)
    batch_tc = jnp.concatenate(
        [batch[_NS:], jnp.full((_NTPAD - (_N - _NS),), _SEGS, jnp.int32)])
    tc_sums, tc_cnt = _tc_partial(x, batch_tc.reshape(_NTB, 1, _BT), W)
    out2 = _combine(partials, cnt_sc, tc_sums, tc_cnt, W, b.reshape(2, 1))
    return out2.T


# speedup vs baseline: 1.7190x; 1.1570x over previous
import functools

import jax
import jax.numpy as jnp
from jax import lax
from jax.experimental import pallas as pl
from jax.experimental.pallas import tpu as pltpu
from jax.experimental.pallas import tpu_sc as plsc

_SEGS = 64
_N = 50000
_D = 1024

_NW = 32
_CH = 1568
_CHL = _N - 31 * _CH
_KW = _CH // 16
_KWL = _CHL // 16

_BT = 2000
_NTB = _N // _BT


_sc_mesh = plsc.VectorSubcoreMesh(core_axis_name="c", subcore_axis_name="s")


@functools.partial(
    pl.kernel,
    out_type=jax.ShapeDtypeStruct((_NW * _SEGS,), jnp.float32),
    mesh=_sc_mesh,
    compiler_params=pltpu.CompilerParams(needs_layout_passes=False),
    scratch_types=[
        pltpu.VMEM((_CH,), jnp.int32),
        pltpu.VMEM((_SEGS,), jnp.float32),
    ],
)
def _sc_counts(batch_hbm, cnt_hbm, bv_v, cnt_v):
    cid = lax.axis_index("c")
    sid = lax.axis_index("s")
    wid = cid * 16 + sid
    base = wid * _CH
    last = wid == _NW - 1

    @pl.when(jnp.logical_not(last))
    def _():
        pltpu.sync_copy(batch_hbm.at[pl.ds(base, _CH)], bv_v)

    @pl.when(last)
    def _():
        pltpu.sync_copy(batch_hbm.at[pl.ds(31 * _CH, _CHL)],
                        bv_v.at[pl.ds(0, _CHL)])

    zeros16 = jnp.zeros((16,), jnp.float32)
    for c in range(_SEGS // 16):
        cnt_v[pl.ds(c * 16, 16)] = zeros16

    ones16 = jnp.ones((16,), jnp.float32)

    def body(k, carry):
        vidx = bv_v[pl.ds(k * 16, 16)]
        plsc.addupdate_scatter(cnt_v, [vidx], ones16)
        return carry

    nw = lax.select(last, _KWL, _KW)
    lax.fori_loop(0, nw, body, 0)

    pltpu.sync_copy(cnt_v, cnt_hbm.at[pl.ds(wid * _SEGS, _SEGS)])


def _tc_body(x_ref, batch_ref, w_ref, sums_ref):
    i = pl.program_id(0)

    @pl.when(i == 0)
    def _():
        sums_ref[...] = jnp.zeros_like(sums_ref)

    h2 = lax.dot_general(w_ref[...], x_ref[...], (((1,), (1,)), ((), ())),
                         preferred_element_type=jnp.float32)
    bidx = batch_ref[0]
    seg = lax.broadcasted_iota(jnp.int32, (_SEGS, _BT), 0)
    onehot = (bidx == seg).astype(jnp.float32)
    psum = lax.dot_general(h2, onehot, (((1,), (1,)), ((), ())),
                           preferred_element_type=jnp.float32)
    sums_ref[...] += psum


def _tc_sums(x, batch3, W):
    return pl.pallas_call(
        _tc_body,
        grid=(_NTB,),
        in_specs=[
            pl.BlockSpec((_BT, _D), lambda i: (i, 0)),
            pl.BlockSpec((1, 1, _BT), lambda i: (i, 0, 0)),
            pl.BlockSpec((2, _D), lambda i: (0, 0)),
        ],
        out_specs=pl.BlockSpec((2, _SEGS), lambda i: (0, 0)),
        out_shape=jax.ShapeDtypeStruct((2, _SEGS), jnp.float32),
        compiler_params=pltpu.CompilerParams(
            dimension_semantics=("arbitrary",)),
    )(x, batch3, W)


def _combine_body(cntsc_ref, tcs_ref, b_ref, out_ref):
    c = cntsc_ref[pl.ds(0, _SEGS)]
    for t in range(1, _NW):
        c = c + cntsc_ref[pl.ds(t * _SEGS, _SEGS)]
    cnt = c[None, :]
    out_ref[...] = (tcs_ref[...] + cnt * b_ref[...]) / jnp.maximum(cnt, 1.0)


def _combine(cnt_sc, tc_sums, b2):
    return pl.pallas_call(
        _combine_body,
        in_specs=[
            pl.BlockSpec((_NW * _SEGS,), lambda: (0,)),
            pl.BlockSpec((2, _SEGS), lambda: (0, 0)),
            pl.BlockSpec((2, 1), lambda: (0, 0)),
        ],
        out_specs=pl.BlockSpec((2, _SEGS), lambda: (0, 0)),
        out_shape=jax.ShapeDtypeStruct((2, _SEGS), jnp.float32),
    )(cnt_sc, tc_sums, b2)


def kernel(x, edge_index, batch, W, b):
    cnt_sc = _sc_counts(batch)
    tc_sums = _tc_sums(x, batch.reshape(_NTB, 1, _BT), W)
    out2 = _combine(cnt_sc, tc_sums, b.reshape(2, 1))
    return out2.T


# speedup vs baseline: 1.7694x; 1.0293x over previous
import functools

import jax
import jax.numpy as jnp
from jax import lax
from jax.experimental import pallas as pl
from jax.experimental.pallas import tpu as pltpu
from jax.experimental.pallas import tpu_sc as plsc

_SEGS = 64
_N = 50000
_D = 1024

_NW = 32
_CH = 1568
_CHL = _N - 31 * _CH
_KW = _CH // 16
_KWL = _CHL // 16

_BT = 2048
_NTB = -(-_N // _BT)


_sc_mesh = plsc.VectorSubcoreMesh(core_axis_name="c", subcore_axis_name="s")


@functools.partial(
    pl.kernel,
    out_type=jax.ShapeDtypeStruct((_NW * _SEGS,), jnp.float32),
    mesh=_sc_mesh,
    compiler_params=pltpu.CompilerParams(needs_layout_passes=False),
    scratch_types=[
        pltpu.VMEM((_CH,), jnp.int32),
        pltpu.VMEM((_SEGS,), jnp.float32),
    ],
)
def _sc_counts(batch_hbm, cnt_hbm, bv_v, cnt_v):
    cid = lax.axis_index("c")
    sid = lax.axis_index("s")
    wid = cid * 16 + sid
    base = wid * _CH
    last = wid == _NW - 1

    @pl.when(jnp.logical_not(last))
    def _():
        pltpu.sync_copy(batch_hbm.at[pl.ds(base, _CH)], bv_v)

    @pl.when(last)
    def _():
        pltpu.sync_copy(batch_hbm.at[pl.ds(31 * _CH, _CHL)],
                        bv_v.at[pl.ds(0, _CHL)])

    zeros16 = jnp.zeros((16,), jnp.float32)
    for c in range(_SEGS // 16):
        cnt_v[pl.ds(c * 16, 16)] = zeros16

    ones16 = jnp.ones((16,), jnp.float32)

    def body(k, carry):
        vidx = bv_v[pl.ds(k * 16, 16)]
        plsc.addupdate_scatter(cnt_v, [vidx], ones16)
        return carry

    nw = lax.select(last, _KWL, _KW)
    lax.fori_loop(0, nw, body, 0)

    pltpu.sync_copy(cnt_v, cnt_hbm.at[pl.ds(wid * _SEGS, _SEGS)])


def _tc_body(x_ref, batch_ref, w_ref, sums_ref):
    i = pl.program_id(0)

    @pl.when(i == 0)
    def _():
        sums_ref[...] = jnp.zeros_like(sums_ref)

    h2 = lax.dot_general(w_ref[...], x_ref[...], (((1,), (1,)), ((), ())),
                         preferred_element_type=jnp.float32)
    row = lax.broadcasted_iota(jnp.int32, (1, _BT), 1) + i * _BT
    h2 = jnp.where(row < _N, h2, 0.0)
    bidx = batch_ref[...][None, :]
    seg = lax.broadcasted_iota(jnp.int32, (_SEGS, _BT), 0)
    onehot = (bidx == seg).astype(jnp.float32)
    psum = lax.dot_general(h2, onehot, (((1,), (1,)), ((), ())),
                           preferred_element_type=jnp.float32)
    sums_ref[...] += psum


def _tc_sums(x, batch, W):
    return pl.pallas_call(
        _tc_body,
        grid=(_NTB,),
        in_specs=[
            pl.BlockSpec((_BT, _D), lambda i: (i, 0)),
            pl.BlockSpec((_BT,), lambda i: (i,)),
            pl.BlockSpec((2, _D), lambda i: (0, 0)),
        ],
        out_specs=pl.BlockSpec((2, _SEGS), lambda i: (0, 0)),
        out_shape=jax.ShapeDtypeStruct((2, _SEGS), jnp.float32),
        compiler_params=pltpu.CompilerParams(
            dimension_semantics=("arbitrary",)),
    )(x, batch, W)


def _combine_body(cntsc_ref, tcs_ref, b_ref, out_ref):
    c = cntsc_ref[pl.ds(0, _SEGS)]
    for t in range(1, _NW):
        c = c + cntsc_ref[pl.ds(t * _SEGS, _SEGS)]
    cnt = c[None, :]
    out_ref[...] = (tcs_ref[...] + cnt * b_ref[...]) / jnp.maximum(cnt, 1.0)


def _combine(cnt_sc, tc_sums, b2):
    return pl.pallas_call(
        _combine_body,
        in_specs=[
            pl.BlockSpec((_NW * _SEGS,), lambda: (0,)),
            pl.BlockSpec((2, _SEGS), lambda: (0, 0)),
            pl.BlockSpec((2, 1), lambda: (0, 0)),
        ],
        out_specs=pl.BlockSpec((2, _SEGS), lambda: (0, 0)),
        out_shape=jax.ShapeDtypeStruct((2, _SEGS), jnp.float32),
    )(cnt_sc, tc_sums, b2)


def kernel(x, edge_index, batch, W, b):
    cnt_sc = _sc_counts(batch)
    tc_sums = _tc_sums(x, batch, W)
    out2 = _combine(cnt_sc, tc_sums, b.reshape(2, 1))
    return out2.T
